# Initial kernel scaffold; baseline (speedup 1.0000x reference)
import jax
import jax.numpy as jnp
from jax.experimental import pallas as pl


def kernel(logits):
    raise NotImplementedError("write your pallas kernel here")


# speedup vs baseline: 106.8777x; 106.8777x over previous
import jax
import jax.numpy as jnp
from jax.experimental import pallas as pl
from jax.experimental.pallas import tpu as pltpu

_N = 4096
_R = 64
_B = 512
_ALIGN = 128
_W = _N + _ALIGN


def _unpack_softmax_kernel(l0_ref, l1_ref, l2_ref, L_ref, T_ref,
                           raw, aligned, sem):
    i0 = pl.program_id(0) * _R
    copies = []
    for r in range(_R):
        i = i0 + r
        s = (i * (i - 1)) // 2
        m = jax.lax.rem(s, _ALIGN)
        a = pl.multiple_of(s - m, _ALIGN)
        for c, src in enumerate((l0_ref, l1_ref, l2_ref)):
            cp = pltpu.make_async_copy(
                src.at[:, pl.ds(a, _W)], raw.at[pl.ds(3 * r + c, 1), :], sem
            )
            cp.start()
            copies.append(cp)
    for cp in copies:
        cp.wait()
    for r in range(_R):
        i = i0 + r
        s = (i * (i - 1)) // 2
        m = jax.lax.rem(s, _ALIGN)
        shift = jax.lax.rem(_W - m, _W)
        rolled = pltpu.roll(raw[pl.ds(3 * r, 3), :], shift, axis=1)
        aligned[:, pl.ds(r, 1), :] = rolled[:, :_N].reshape(3, 1, _N)
    e0 = jnp.exp(aligned[0])
    e1 = jnp.exp(aligned[1])
    e2 = jnp.exp(aligned[2])
    d = e0 + e1 + e2
    rows = jax.lax.broadcasted_iota(jnp.int32, (_R, _N), 0) + i0
    cols = jax.lax.broadcasted_iota(jnp.int32, (_R, _N), 1)
    mask = cols < rows
    zero = jnp.zeros((), jnp.float32)
    L_ref[...] = jnp.where(mask, e0 / d, zero)
    T_ref[...] = jnp.where(mask, e1 / d, zero)


def _combine_kernel(l_ref, t_ref, o_ref):
    o_ref[...] = l_ref[...] + t_ref[...].T


def kernel(logits):
    n = _N
    flat = jnp.pad(logits, ((0, 0), (0, _W)))
    l0 = flat[0].reshape(1, -1)
    l1 = flat[1].reshape(1, -1)
    l2 = flat[2].reshape(1, -1)

    L, T = pl.pallas_call(
        _unpack_softmax_kernel,
        grid=(n // _R,),
        in_specs=[pl.BlockSpec(memory_space=pltpu.MemorySpace.HBM)] * 3,
        out_specs=[pl.BlockSpec((_R, n), lambda I: (I, 0))] * 2,
        out_shape=[jax.ShapeDtypeStruct((n, n), jnp.float32)] * 2,
        scratch_shapes=[
            pltpu.VMEM((3 * _R, _W), jnp.float32),
            pltpu.VMEM((3, _R, _N), jnp.float32),
            pltpu.SemaphoreType.DMA,
        ],
    )(l0, l1, l2)

    out = pl.pallas_call(
        _combine_kernel,
        grid=(n // _B, n // _B),
        in_specs=[
            pl.BlockSpec((_B, _B), lambda i, j: (i, j)),
            pl.BlockSpec((_B, _B), lambda i, j: (j, i)),
        ],
        out_specs=pl.BlockSpec((_B, _B), lambda i, j: (i, j)),
        out_shape=jax.ShapeDtypeStruct((n, n), jnp.float32),
    )(L, T)
    return out


# speedup vs baseline: 159.4242x; 1.4916x over previous
import jax
import jax.numpy as jnp
from jax.experimental import pallas as pl
from jax.experimental.pallas import tpu as pltpu

_N = 4096
_R = 64
_B = 512
_ALIGN = 128
_W = _N + _ALIGN


def _unpack_softmax_kernel(flat_ref, L_ref, T_ref, raw, aligned, sems):
    I = pl.program_id(0)
    nb = _N // _R

    def copies(blk, slot):
        cps = []
        for r in range(_R):
            i = blk * _R + r
            s = (i * (i - 1)) // 2
            m = jax.lax.rem(s, _ALIGN)
            a = pl.multiple_of(s - m, _ALIGN)
            cps.append(pltpu.make_async_copy(
                flat_ref.at[:, :, pl.ds(a, _W)],
                raw.at[slot, pl.ds(r, 1)],
                sems.at[slot],
            ))
        return cps

    slot = jax.lax.rem(I, 2)

    @pl.when(I == 0)
    def _():
        for cp in copies(I, slot):
            cp.start()

    @pl.when(I + 1 < nb)
    def _():
        for cp in copies(I + 1, 1 - slot):
            cp.start()

    for cp in copies(I, slot):
        cp.wait()

    i0 = I * _R
    for r in range(_R):
        i = i0 + r
        s = (i * (i - 1)) // 2
        m = jax.lax.rem(s, _ALIGN)
        shift = jax.lax.rem(_W - m, _W)
        rolled = pltpu.roll(raw[slot, r], shift, axis=1)
        aligned[:, pl.ds(r, 1), :] = rolled[:, :_N].reshape(3, 1, _N)
    e0 = jnp.exp(aligned[0])
    e1 = jnp.exp(aligned[1])
    e2 = jnp.exp(aligned[2])
    d = e0 + e1 + e2
    rows = jax.lax.broadcasted_iota(jnp.int32, (_R, _N), 0) + i0
    cols = jax.lax.broadcasted_iota(jnp.int32, (_R, _N), 1)
    mask = cols < rows
    zero = jnp.zeros((), jnp.float32)
    L_ref[...] = jnp.where(mask, e0 / d, zero)
    T_ref[...] = jnp.where(mask, e1 / d, zero)


def _combine_kernel(l_ref, t_ref, o_ref):
    o_ref[...] = l_ref[...] + t_ref[...].T


def kernel(logits):
    n = _N
    flat = jnp.pad(logits, ((0, 0), (0, _W)))[None]

    L, T = pl.pallas_call(
        _unpack_softmax_kernel,
        grid=(n // _R,),
        in_specs=[pl.BlockSpec(memory_space=pltpu.MemorySpace.HBM)],
        out_specs=[pl.BlockSpec((_R, n), lambda I: (I, 0))] * 2,
        out_shape=[jax.ShapeDtypeStruct((n, n), jnp.float32)] * 2,
        scratch_shapes=[
            pltpu.VMEM((2, _R, 3, _W), jnp.float32),
            pltpu.VMEM((3, _R, _N), jnp.float32),
            pltpu.SemaphoreType.DMA((2,)),
        ],
    )(flat)

    out = pl.pallas_call(
        _combine_kernel,
        grid=(n // _B, n // _B),
        in_specs=[
            pl.BlockSpec((_B, _B), lambda i, j: (i, j)),
            pl.BlockSpec((_B, _B), lambda i, j: (j, i)),
        ],
        out_specs=pl.BlockSpec((_B, _B), lambda i, j: (i, j)),
        out_shape=jax.ShapeDtypeStruct((n, n), jnp.float32),
    )(L, T)
    return out


# speedup vs baseline: 171.7762x; 1.0775x over previous
import jax
import jax.numpy as jnp
from jax.experimental import pallas as pl
from jax.experimental.pallas import tpu as pltpu

_N = 4096
_R = 64
_B = 512
_ALIGN = 128
_W = _N + _ALIGN


def _unpack_softmax_kernel(flat_ref, L_ref, T_ref, raw, aligned, sems):
    I = pl.program_id(0)
    nb = _N // _R

    def copies(blk, slot):
        cps = []
        for r in range(_R):
            i = blk * _R + r
            s = (i * (i - 1)) // 2
            m = jax.lax.rem(s, _ALIGN)
            a = pl.multiple_of(s - m, _ALIGN)
            cps.append(pltpu.make_async_copy(
                flat_ref.at[:, :, pl.ds(a, _W)],
                raw.at[slot, pl.ds(r, 1)],
                sems.at[slot],
            ))
        return cps

    slot = jax.lax.rem(I, 2)

    @pl.when(I == 0)
    def _():
        for cp in copies(I, slot):
            cp.start()

    @pl.when(I + 1 < nb)
    def _():
        for cp in copies(I + 1, 1 - slot):
            cp.start()

    for cp in copies(I, slot):
        cp.wait()

    i0 = I * _R
    for r in range(_R):
        i = i0 + r
        s = (i * (i - 1)) // 2
        m = jax.lax.rem(s, _ALIGN)
        shift = jax.lax.rem(_W - m, _W)
        rolled = pltpu.roll(raw[slot, r], shift, axis=1)
        aligned[:, pl.ds(r, 1), :] = rolled[:, :_N].reshape(3, 1, _N)
    e0 = jnp.exp(aligned[0])
    e1 = jnp.exp(aligned[1])
    e2 = jnp.exp(aligned[2])
    d = e0 + e1 + e2
    rows = jax.lax.broadcasted_iota(jnp.int32, (_R, _N), 0) + i0
    cols = jax.lax.broadcasted_iota(jnp.int32, (_R, _N), 1)
    mask = cols < rows
    zero = jnp.zeros((), jnp.float32)
    L_ref[...] = jnp.where(mask, e0 / d, zero)
    T_ref[...] = jnp.where(mask, e1 / d, zero)


def _combine_kernel(l_ref, t_ref, o_ref):
    o_ref[...] = l_ref[...] + t_ref[...].T


_NT = _N // _B


def _tri_decode(t):
    i = jnp.int32(0)
    for k in range(1, _NT):
        cum_k = k * _NT - k * (k - 1) // 2
        i = i + jnp.where(t >= cum_k, 1, 0).astype(jnp.int32)
    cum_i = i * _NT - i * (i - 1) // 2
    j = i + (t - cum_i)
    return i, j


def _ij_map(t):
    i, j = _tri_decode(t)
    return i, j


def _ji_map(t):
    i, j = _tri_decode(t)
    return j, i


def kernel(logits):
    n = _N
    flat = jnp.pad(logits, ((0, 0), (0, _W)))[None]

    L, T = pl.pallas_call(
        _unpack_softmax_kernel,
        grid=(n // _R,),
        in_specs=[pl.BlockSpec(memory_space=pltpu.MemorySpace.HBM)],
        out_specs=[pl.BlockSpec((_R, n), lambda I: (I, 0))] * 2,
        out_shape=[jax.ShapeDtypeStruct((n, n), jnp.float32)] * 2,
        scratch_shapes=[
            pltpu.VMEM((2, _R, 3, _W), jnp.float32),
            pltpu.VMEM((3, _R, _N), jnp.float32),
            pltpu.SemaphoreType.DMA((2,)),
        ],
    )(flat)

    n_tri = _NT * (_NT + 1) // 2
    out = pl.pallas_call(
        _combine_kernel,
        grid=(n_tri,),
        in_specs=[
            pl.BlockSpec((_B, _B), _ij_map),
            pl.BlockSpec((_B, _B), _ji_map),
        ],
        out_specs=pl.BlockSpec((_B, _B), _ij_map),
        out_shape=jax.ShapeDtypeStruct((n, n), jnp.float32),
        input_output_aliases={0: 0},
    )(L, T)
    return out


# speedup vs baseline: 182.4401x; 1.0621x over previous
import jax
import jax.numpy as jnp
from jax.experimental import pallas as pl
from jax.experimental.pallas import tpu as pltpu

_N = 4096
_R = 64
_B = 512
_ALIGN = 128
_W = _N + _ALIGN


_TIERS = 4
_BLOCKS_PER_TIER = _N // _R // _TIERS
_TIER_W = [(_N // _TIERS) * (t + 1) + _ALIGN for t in range(_TIERS)]


def _unpack_softmax_kernel(flat_ref, L_ref, T_ref, raw, aligned, sems):
    I = pl.program_id(0)
    nb = _N // _R

    def for_copies(blk, slot, fn):
        tier = jax.lax.div(blk, _BLOCKS_PER_TIER)
        for t in range(_TIERS):
            w = _TIER_W[t]

            @pl.when(tier == t)
            def _():
                for r in range(_R):
                    i = blk * _R + r
                    s = (i * (i - 1)) // 2
                    m = jax.lax.rem(s, _ALIGN)
                    a = pl.multiple_of(s - m, _ALIGN)
                    fn(pltpu.make_async_copy(
                        flat_ref.at[:, :, pl.ds(a, w)],
                        raw.at[slot, pl.ds(r, 1), :, pl.ds(0, w)],
                        sems.at[slot],
                    ))

    slot = jax.lax.rem(I, 2)

    @pl.when(I == 0)
    def _():
        for_copies(I, slot, lambda cp: cp.start())

    @pl.when(I + 1 < nb)
    def _():
        for_copies(I + 1, 1 - slot, lambda cp: cp.start())

    for_copies(I, slot, lambda cp: cp.wait())

    i0 = I * _R
    for r in range(_R):
        i = i0 + r
        s = (i * (i - 1)) // 2
        m = jax.lax.rem(s, _ALIGN)
        shift = jax.lax.rem(_W - m, _W)
        rolled = pltpu.roll(raw[slot, r], shift, axis=1)
        aligned[:, pl.ds(r, 1), :] = rolled[:, :_N].reshape(3, 1, _N)
    e0 = jnp.exp(aligned[0])
    e1 = jnp.exp(aligned[1])
    e2 = jnp.exp(aligned[2])
    d = e0 + e1 + e2
    rows = jax.lax.broadcasted_iota(jnp.int32, (_R, _N), 0) + i0
    cols = jax.lax.broadcasted_iota(jnp.int32, (_R, _N), 1)
    mask = cols < rows
    zero = jnp.zeros((), jnp.float32)
    L_ref[...] = jnp.where(mask, e0 / d, zero)
    T_ref[...] = jnp.where(mask, e1 / d, zero)


def _combine_kernel(l_ref, t_ref, o_ref):
    o_ref[...] = l_ref[...] + t_ref[...].T


_NT = _N // _B


def _tri_decode(t):
    i = jnp.int32(0)
    for k in range(1, _NT):
        cum_k = k * _NT - k * (k - 1) // 2
        i = i + jnp.where(t >= cum_k, 1, 0).astype(jnp.int32)
    cum_i = i * _NT - i * (i - 1) // 2
    j = i + (t - cum_i)
    return i, j


def _ij_map(t):
    i, j = _tri_decode(t)
    return i, j


def _ji_map(t):
    i, j = _tri_decode(t)
    return j, i


def kernel(logits):
    n = _N
    flat = jnp.pad(logits, ((0, 0), (0, _W)))[None]

    L, T = pl.pallas_call(
        _unpack_softmax_kernel,
        grid=(n // _R,),
        in_specs=[pl.BlockSpec(memory_space=pltpu.MemorySpace.HBM)],
        out_specs=[pl.BlockSpec((_R, n), lambda I: (I, 0))] * 2,
        out_shape=[jax.ShapeDtypeStruct((n, n), jnp.float32)] * 2,
        scratch_shapes=[
            pltpu.VMEM((2, _R, 3, _W), jnp.float32),
            pltpu.VMEM((3, _R, _N), jnp.float32),
            pltpu.SemaphoreType.DMA((2,)),
        ],
    )(flat)

    n_tri = _NT * (_NT + 1) // 2
    out = pl.pallas_call(
        _combine_kernel,
        grid=(n_tri,),
        in_specs=[
            pl.BlockSpec((_B, _B), _ij_map),
            pl.BlockSpec((_B, _B), _ji_map),
        ],
        out_specs=pl.BlockSpec((_B, _B), _ij_map),
        out_shape=jax.ShapeDtypeStruct((n, n), jnp.float32),
        input_output_aliases={0: 0},
    )(L, T)
    return out


# speedup vs baseline: 237.8954x; 1.3040x over previous
import jax
import jax.numpy as jnp
from jax.experimental import pallas as pl
from jax.experimental.pallas import tpu as pltpu

_N = 4096
_R = 64
_B = 512
_ALIGN = 128
_W = _N + _ALIGN


_TIERS = 4
_BLOCKS_PER_TIER = _N // _R // _TIERS
_TIER_W = [(_N // _TIERS) * (t + 1) + _ALIGN for t in range(_TIERS)]


def _unpack_softmax_kernel(flat_ref, L_ref, T_ref, raw, aligned, sems):
    I = pl.program_id(0)
    nb = _N // _R

    def for_copies(blk, slot, fn):
        tier = jax.lax.div(blk, _BLOCKS_PER_TIER)
        for t in range(_TIERS):
            w = _TIER_W[t]

            @pl.when(tier == t)
            def _():
                for r in range(_R):
                    i = blk * _R + r
                    s = (i * (i - 1)) // 2
                    m = jax.lax.rem(s, _ALIGN)
                    a = pl.multiple_of(s - m, _ALIGN)
                    fn(pltpu.make_async_copy(
                        flat_ref.at[:, :, pl.ds(a, w)],
                        raw.at[slot, pl.ds(r, 1), :, pl.ds(0, w)],
                        sems.at[slot],
                    ))

    slot = jax.lax.rem(I, 2)

    @pl.when(I == 0)
    def _():
        for_copies(I, slot, lambda cp: cp.start())

    @pl.when(I + 1 < nb)
    def _():
        for_copies(I + 1, 1 - slot, lambda cp: cp.start())

    for_copies(I, slot, lambda cp: cp.wait())

    i0 = I * _R
    tier = jax.lax.div(I, _BLOCKS_PER_TIER)
    zero = jnp.zeros((), jnp.float32)
    for t in range(_TIERS):
        w = _TIER_W[t]
        wd = w - _ALIGN

        @pl.when(tier == t)
        def _():
            for r in range(_R):
                i = i0 + r
                s = (i * (i - 1)) // 2
                m = jax.lax.rem(s, _ALIGN)
                shift = jax.lax.rem(w - m, w)
                rolled = pltpu.roll(raw[slot, r, :, :w], shift, axis=1)
                aligned[:, pl.ds(r, 1), :wd] = (
                    rolled[:, :wd].reshape(3, 1, wd))
            e0 = jnp.exp(aligned[0, :, :wd])
            e1 = jnp.exp(aligned[1, :, :wd])
            e2 = jnp.exp(aligned[2, :, :wd])
            d = e0 + e1 + e2
            rows = jax.lax.broadcasted_iota(jnp.int32, (_R, wd), 0) + i0
            cols = jax.lax.broadcasted_iota(jnp.int32, (_R, wd), 1)
            mask = cols < rows
            L_ref[:, :wd] = jnp.where(mask, e0 / d, zero)
            T_ref[:, :wd] = jnp.where(mask, e1 / d, zero)
            if wd < _N:
                zeros_tail = jnp.zeros((_R, _N - wd), jnp.float32)
                L_ref[:, wd:] = zeros_tail
                T_ref[:, wd:] = zeros_tail


def _diag_kernel(l_ref, t_ref, o_ref):
    o_ref[...] = l_ref[...] + t_ref[...].T


def _upper_kernel(prev_ref, t_ref, o_ref):
    del prev_ref
    o_ref[...] = t_ref[...].T


_NT = _N // _B


def _upper_decode(t):
    i = jnp.int32(0)
    for k in range(1, _NT - 1):
        cum_k = k * (_NT - 1) - k * (k - 1) // 2
        i = i + jnp.where(t >= cum_k, 1, 0).astype(jnp.int32)
    cum_i = i * (_NT - 1) - i * (i - 1) // 2
    j = i + 1 + (t - cum_i)
    return i, j


def _upper_ij_map(t):
    i, j = _upper_decode(t)
    return i, j


def _upper_ji_map(t):
    i, j = _upper_decode(t)
    return j, i


def kernel(logits):
    n = _N
    flat = jnp.pad(logits, ((0, 0), (0, _W)))[None]

    L, T = pl.pallas_call(
        _unpack_softmax_kernel,
        grid=(n // _R,),
        in_specs=[pl.BlockSpec(memory_space=pltpu.MemorySpace.HBM)],
        out_specs=[pl.BlockSpec((_R, n), lambda I: (I, 0))] * 2,
        out_shape=[jax.ShapeDtypeStruct((n, n), jnp.float32)] * 2,
        scratch_shapes=[
            pltpu.VMEM((2, _R, 3, _W), jnp.float32),
            pltpu.VMEM((3, _R, _N), jnp.float32),
            pltpu.SemaphoreType.DMA((2,)),
        ],
    )(flat)

    out1 = pl.pallas_call(
        _diag_kernel,
        grid=(_NT,),
        in_specs=[
            pl.BlockSpec((_B, _B), lambda t: (t, t)),
            pl.BlockSpec((_B, _B), lambda t: (t, t)),
        ],
        out_specs=pl.BlockSpec((_B, _B), lambda t: (t, t)),
        out_shape=jax.ShapeDtypeStruct((n, n), jnp.float32),
        input_output_aliases={0: 0},
    )(L, T)

    n_upper = _NT * (_NT - 1) // 2
    out = pl.pallas_call(
        _upper_kernel,
        grid=(n_upper,),
        in_specs=[
            pl.BlockSpec(memory_space=pltpu.MemorySpace.HBM),
            pl.BlockSpec((_B, _B), _upper_ji_map),
        ],
        out_specs=pl.BlockSpec((_B, _B), _upper_ij_map),
        out_shape=jax.ShapeDtypeStruct((n, n), jnp.float32),
        input_output_aliases={0: 0},
    )(out1, T)
    return out


# speedup vs baseline: 249.6690x; 1.0495x over previous
import jax
import jax.numpy as jnp
from jax.experimental import pallas as pl
from jax.experimental.pallas import tpu as pltpu

_N = 4096
_R = 64
_B = 512
_ALIGN = 128
_W = _N + _ALIGN


_TIERS = 8
_BLOCKS_PER_TIER = _N // _R // _TIERS
_TIER_W = [(_N // _TIERS) * (t + 1) + _ALIGN for t in range(_TIERS)]


def _unpack_softmax_kernel(flat_ref, L_ref, T_ref, raw, aligned, sems):
    I = pl.program_id(0)
    nb = _N // _R

    def for_copies(blk, slot, fn):
        tier = jax.lax.div(blk, _BLOCKS_PER_TIER)
        for t in range(_TIERS):
            w = _TIER_W[t]

            @pl.when(tier == t)
            def _():
                for r in range(_R):
                    i = blk * _R + r
                    s = (i * (i - 1)) // 2
                    m = jax.lax.rem(s, _ALIGN)
                    a = pl.multiple_of(s - m, _ALIGN)
                    fn(pltpu.make_async_copy(
                        flat_ref.at[:, :, pl.ds(a, w)],
                        raw.at[slot, pl.ds(r, 1), :, pl.ds(0, w)],
                        sems.at[slot],
                    ))

    slot = jax.lax.rem(I, 2)

    @pl.when(I == 0)
    def _():
        for_copies(I, slot, lambda cp: cp.start())

    @pl.when(I + 1 < nb)
    def _():
        for_copies(I + 1, 1 - slot, lambda cp: cp.start())

    for_copies(I, slot, lambda cp: cp.wait())

    i0 = I * _R
    tier = jax.lax.div(I, _BLOCKS_PER_TIER)
    zero = jnp.zeros((), jnp.float32)
    for t in range(_TIERS):
        w = _TIER_W[t]
        wd = w - _ALIGN

        @pl.when(tier == t)
        def _():
            for r in range(_R):
                i = i0 + r
                s = (i * (i - 1)) // 2
                m = jax.lax.rem(s, _ALIGN)
                shift = jax.lax.rem(w - m, w)
                rolled = pltpu.roll(raw[slot, r, :, :w], shift, axis=1)
                aligned[:, pl.ds(r, 1), :wd] = (
                    rolled[:, :wd].reshape(3, 1, wd))
            e0 = jnp.exp(aligned[0, :, :wd])
            e1 = jnp.exp(aligned[1, :, :wd])
            e2 = jnp.exp(aligned[2, :, :wd])
            d = e0 + e1 + e2
            rows = jax.lax.broadcasted_iota(jnp.int32, (_R, wd), 0) + i0
            cols = jax.lax.broadcasted_iota(jnp.int32, (_R, wd), 1)
            mask = cols < rows
            L_ref[:, :wd] = jnp.where(mask, e0 / d, zero)
            T_ref[:, :wd] = jnp.where(mask, e1 / d, zero)
            if wd < _N:
                zeros_tail = jnp.zeros((_R, _N - wd), jnp.float32)
                L_ref[:, wd:] = zeros_tail
                T_ref[:, wd:] = zeros_tail


def _diag_kernel(l_ref, t_ref, o_ref):
    o_ref[...] = l_ref[...] + t_ref[...].T


def _upper_kernel(prev_ref, t_ref, o_ref):
    del prev_ref
    o_ref[...] = t_ref[...].T


_NT = _N // _B


def _upper_decode(t):
    i = jnp.int32(0)
    for k in range(1, _NT - 1):
        cum_k = k * (_NT - 1) - k * (k - 1) // 2
        i = i + jnp.where(t >= cum_k, 1, 0).astype(jnp.int32)
    cum_i = i * (_NT - 1) - i * (i - 1) // 2
    j = i + 1 + (t - cum_i)
    return i, j


def _upper_ij_map(t):
    i, j = _upper_decode(t)
    return i, j


def _upper_ji_map(t):
    i, j = _upper_decode(t)
    return j, i


def kernel(logits):
    n = _N
    flat = jnp.pad(logits, ((0, 0), (0, _W)))[None]

    L, T = pl.pallas_call(
        _unpack_softmax_kernel,
        grid=(n // _R,),
        in_specs=[pl.BlockSpec(memory_space=pltpu.MemorySpace.HBM)],
        out_specs=[pl.BlockSpec((_R, n), lambda I: (I, 0))] * 2,
        out_shape=[jax.ShapeDtypeStruct((n, n), jnp.float32)] * 2,
        scratch_shapes=[
            pltpu.VMEM((2, _R, 3, _W), jnp.float32),
            pltpu.VMEM((3, _R, _N), jnp.float32),
            pltpu.SemaphoreType.DMA((2,)),
        ],
    )(flat)

    out1 = pl.pallas_call(
        _diag_kernel,
        grid=(_NT,),
        in_specs=[
            pl.BlockSpec((_B, _B), lambda t: (t, t)),
            pl.BlockSpec((_B, _B), lambda t: (t, t)),
        ],
        out_specs=pl.BlockSpec((_B, _B), lambda t: (t, t)),
        out_shape=jax.ShapeDtypeStruct((n, n), jnp.float32),
        input_output_aliases={0: 0},
    )(L, T)

    n_upper = _NT * (_NT - 1) // 2
    out = pl.pallas_call(
        _upper_kernel,
        grid=(n_upper,),
        in_specs=[
            pl.BlockSpec(memory_space=pltpu.MemorySpace.HBM),
            pl.BlockSpec((_B, _B), _upper_ji_map),
        ],
        out_specs=pl.BlockSpec((_B, _B), _upper_ij_map),
        out_shape=jax.ShapeDtypeStruct((n, n), jnp.float32),
        input_output_aliases={0: 0},
    )(out1, T)
    return out
